# baseline (speedup 1.0000x reference)
import functools

import jax
import jax.numpy as jnp
from jax import lax
from jax.experimental import pallas as pl
from jax.experimental.pallas import tpu as pltpu
from jax.experimental.pallas import tpu_sc as plsc

NC = 2
NS = 16
NW = NC * NS
LANES = 16
TILE_W = 128
WC = 1024
SLABW = 1041


def _iota16():
    return lax.broadcasted_iota(jnp.int32, (LANES,), 0)


def _transpose_128(slab, tr1d, t8, col_base, d):
    rows = _iota16()
    iota = _iota16()

    def blk(k, c):
        c0 = col_base + k * 32
        vals = []
        for b in range(32):
            colv = jnp.broadcast_to(c0 + b, (LANES,)).astype(jnp.int32)
            vals.append(plsc.load_gather(slab, [rows, colv]))
        for b in range(32):
            addr = (c0 + b) * d + iota
            plsc.store_scatter(tr1d, [addr], vals[b])
        return c
    lax.fori_loop(0, TILE_W // 32, blk, 0)


@functools.lru_cache(maxsize=None)
def _build_detile(V: int, D: int):
    n_full = V // WC
    tail = V - n_full * WC
    tail_full = (tail // TILE_W) * TILE_W
    tail_rem = tail - tail_full
    base_wins = n_full // NW
    extra = n_full - base_wins * NW
    mesh = plsc.VectorSubcoreMesh(core_axis_name="c", subcore_axis_name="s")

    @functools.partial(
        pl.kernel,
        mesh=mesh,
        out_type=jax.ShapeDtypeStruct((V * D,), jnp.float32),
        scratch_types=[
            pltpu.VMEM((LANES, SLABW), jnp.float32),
            pltpu.VMEM((LANES, SLABW), jnp.float32),
            pltpu.VMEM((WC * D,), jnp.float32),
            pltpu.VMEM((WC * D,), jnp.float32),
            pltpu.SemaphoreType.DMA,
            pltpu.SemaphoreType.DMA,
            pltpu.SemaphoreType.DMA,
            pltpu.SemaphoreType.DMA,
        ],
        compiler_params=pltpu.CompilerParams(needs_layout_passes=False),
    )
    def detile_kernel(wt_hbm, wtail_hbm, g_hbm, slab0, slab1, tr0, tr1,
                      si0, si1, so0, so1):
        wid = lax.axis_index("s") * NC + lax.axis_index("c")
        n_win = base_wins + jnp.where(wid < extra, 1, 0)
        slabs = [slab0, slab1]
        trs = [tr0, tr1]
        sis = [si0, si1]
        sos = [so0, so1]

        def v0_of(g):
            return pl.multiple_of((wid + g * NW) * WC, WC)

        def start_in(g, s):
            pltpu.async_copy(wt_hbm.at[:, pl.ds(v0_of(g), WC)],
                             slabs[s].at[:, pl.ds(0, WC)], sis[s])

        def wait_in(g, s):
            pltpu.make_async_copy(wt_hbm.at[:, pl.ds(v0_of(g), WC)],
                                  slabs[s].at[:, pl.ds(0, WC)],
                                  sis[s]).wait()

        def g0_of(g):
            return pl.multiple_of(v0_of(g) * D, WC * D)

        def start_out(g, s):
            pltpu.async_copy(trs[s], g_hbm.at[pl.ds(g0_of(g), WC * D)],
                             sos[s])

        def wait_out(g, s):
            pltpu.make_async_copy(trs[s],
                                  g_hbm.at[pl.ds(g0_of(g), WC * D)],
                                  sos[s]).wait()

        @pl.when(n_win > 0)
        def _():
            start_in(0, 0)

        def body(g, carry):
            for s in range(2):
                @pl.when(lax.rem(g, 2) == s)
                def _():
                    @pl.when(g + 1 < n_win)
                    def _():
                        start_in(g + 1, 1 - s)

                    @pl.when(g >= 2)
                    def _():
                        wait_out(g - 2, s)

                    wait_in(g, s)

                    def tb(t8, c):
                        _transpose_128(slabs[s], trs[s], t8, t8 * TILE_W, D)
                        return c
                    lax.fori_loop(0, WC // TILE_W, tb, 0)
                    start_out(g, s)
            return carry

        lax.fori_loop(0, n_win, body, 0)

        for d in (2, 1):
            @pl.when(n_win >= d)
            def _():
                g = n_win - d
                for s in range(2):
                    @pl.when(lax.rem(g, 2) == s)
                    def _():
                        wait_out(g, s)

        @pl.when(wid == NW - 1)
        def _():
            v0 = n_full * WC
            pltpu.sync_copy(wt_hbm.at[:, pl.ds(v0, tail_full)],
                            slab0.at[:, pl.ds(0, tail_full)])

            def tb(t8, c):
                _transpose_128(slab0, tr0, t8, t8 * TILE_W, D)
                return c
            lax.fori_loop(0, tail_full // TILE_W, tb, 0)
            pltpu.sync_copy(tr0.at[pl.ds(0, tail_full * D)],
                            g_hbm.at[pl.ds(v0 * D, tail_full * D)])

        @pl.when(wid == NW - 2)
        def _():
            nrem = tail_rem * D
            pltpu.sync_copy(wtail_hbm, tr1.at[pl.ds(0, nrem)])
            pltpu.sync_copy(
                tr1.at[pl.ds(0, nrem)],
                g_hbm.at[pl.ds((n_full * WC + tail_full) * D, nrem)])

    return detile_kernel


@functools.lru_cache(maxsize=None)
def _build_gather(B: int, B_pad: int, V: int, D: int):
    b_per_w = B_pad // NW
    last_w = B - (NW - 1) * b_per_w
    assert b_per_w % 8 == 0 and 0 < last_w <= b_per_w
    mesh = plsc.VectorSubcoreMesh(core_axis_name="c", subcore_axis_name="s")

    @functools.partial(
        pl.kernel,
        mesh=mesh,
        out_type=jax.ShapeDtypeStruct((B, D), jnp.float32),
        scratch_types=[
            pltpu.VMEM((b_per_w,), jnp.int32),
            pltpu.VMEM((b_per_w, D), jnp.float32),
            pltpu.SemaphoreType.DMA,
        ],
        compiler_params=pltpu.CompilerParams(use_tc_tiling_on_sc=False),
    )
    def gather_kernel(idx_hbm, table_hbm, out_hbm, idx_v, rows_v, sem):
        wid = lax.axis_index("s") * NC + lax.axis_index("c")
        base = pl.multiple_of(wid * b_per_w, 8)
        pltpu.sync_copy(idx_hbm.at[pl.ds(base, b_per_w)], idx_v)
        pltpu.async_copy(table_hbm.at[idx_v], rows_v, sem).wait()

        @pl.when(wid < NW - 1)
        def _():
            pltpu.sync_copy(rows_v, out_hbm.at[pl.ds(base, b_per_w), :])

        @pl.when(wid == NW - 1)
        def _():
            pltpu.sync_copy(rows_v.at[pl.ds(0, last_w), :],
                            out_hbm.at[pl.ds(base, last_w), :])

    return gather_kernel


def kernel(atom_types, W):
    idx = jnp.squeeze(atom_types).astype(jnp.int32)
    B = idx.shape[0]
    V, D = W.shape
    align = 16 * NW
    B_pad = ((B + align - 1) // align) * align
    if B_pad != B:
        pad = (jnp.arange(B_pad - B, dtype=jnp.int32) * 997) % V
        idx = jnp.concatenate([idx, pad])
    n_full = V // WC
    tail_full = ((V - n_full * WC) // TILE_W) * TILE_W
    v_aligned = n_full * WC + tail_full
    wtail = W[v_aligned:].reshape(-1)
    table1d = _build_detile(V, D)(W.T, wtail)
    return _build_gather(B, B_pad, V, D)(idx, table1d.reshape(V, D))


# speedup vs baseline: 1.0487x; 1.0487x over previous
import functools

import jax
import jax.numpy as jnp
from jax import lax
from jax.experimental import pallas as pl
from jax.experimental.pallas import tpu as pltpu
from jax.experimental.pallas import tpu_sc as plsc

NC = 2
NS = 16
NW = NC * NS
LANES = 16
TILE_W = 128
WC = 1024
SLABW = 1041


def _iota16():
    return lax.broadcasted_iota(jnp.int32, (LANES,), 0)


def _transpose_128(slab, tr1d, t8, col_base, d):
    rows = _iota16()
    iota = _iota16()

    def gathers(k):
        c0 = col_base + k * 16
        return [
            plsc.load_gather(
                slab,
                [rows, jnp.broadcast_to(c0 + b, (LANES,)).astype(jnp.int32)])
            for b in range(16)
        ]

    def stores(k, vals):
        c0 = col_base + k * 16
        for b in range(16):
            plsc.store_scatter(tr1d, [(c0 + b) * d + iota], vals[b])

    def blk(k, vals_prev):
        vals = gathers(k)
        stores(k - 1, vals_prev)
        return vals
    last = lax.fori_loop(1, TILE_W // 16, blk, gathers(0))
    stores(TILE_W // 16 - 1, last)


@functools.lru_cache(maxsize=None)
def _build_detile(V: int, D: int):
    n_full = V // WC
    tail = V - n_full * WC
    tail_full = (tail // TILE_W) * TILE_W
    tail_rem = tail - tail_full
    base_wins = n_full // NW
    extra = n_full - base_wins * NW
    mesh = plsc.VectorSubcoreMesh(core_axis_name="c", subcore_axis_name="s")

    @functools.partial(
        pl.kernel,
        mesh=mesh,
        out_type=jax.ShapeDtypeStruct((V * D,), jnp.float32),
        scratch_types=[
            pltpu.VMEM((LANES, SLABW), jnp.float32),
            pltpu.VMEM((LANES, SLABW), jnp.float32),
            pltpu.VMEM((WC * D,), jnp.float32),
            pltpu.VMEM((WC * D,), jnp.float32),
            pltpu.SemaphoreType.DMA,
            pltpu.SemaphoreType.DMA,
            pltpu.SemaphoreType.DMA,
            pltpu.SemaphoreType.DMA,
        ],
        compiler_params=pltpu.CompilerParams(needs_layout_passes=False),
    )
    def detile_kernel(wt_hbm, wtail_hbm, g_hbm, slab0, slab1, tr0, tr1,
                      si0, si1, so0, so1):
        wid = lax.axis_index("s") * NC + lax.axis_index("c")
        n_win = base_wins + jnp.where(wid < extra, 1, 0)
        slabs = [slab0, slab1]
        trs = [tr0, tr1]
        sis = [si0, si1]
        sos = [so0, so1]

        def v0_of(g):
            return pl.multiple_of((wid + g * NW) * WC, WC)

        def start_in(g, s):
            pltpu.async_copy(wt_hbm.at[:, pl.ds(v0_of(g), WC)],
                             slabs[s].at[:, pl.ds(0, WC)], sis[s])

        def wait_in(g, s):
            pltpu.make_async_copy(wt_hbm.at[:, pl.ds(v0_of(g), WC)],
                                  slabs[s].at[:, pl.ds(0, WC)],
                                  sis[s]).wait()

        def g0_of(g):
            return pl.multiple_of(v0_of(g) * D, WC * D)

        def start_out(g, s):
            pltpu.async_copy(trs[s], g_hbm.at[pl.ds(g0_of(g), WC * D)],
                             sos[s])

        def wait_out(g, s):
            pltpu.make_async_copy(trs[s],
                                  g_hbm.at[pl.ds(g0_of(g), WC * D)],
                                  sos[s]).wait()

        @pl.when(n_win > 0)
        def _():
            start_in(0, 0)

        def body(g, carry):
            for s in range(2):
                @pl.when(lax.rem(g, 2) == s)
                def _():
                    @pl.when(g + 1 < n_win)
                    def _():
                        start_in(g + 1, 1 - s)

                    @pl.when(g >= 2)
                    def _():
                        wait_out(g - 2, s)

                    wait_in(g, s)

                    def tb(t8, c):
                        _transpose_128(slabs[s], trs[s], t8, t8 * TILE_W, D)
                        return c
                    lax.fori_loop(0, WC // TILE_W, tb, 0)
                    start_out(g, s)
            return carry

        lax.fori_loop(0, n_win, body, 0)

        for d in (2, 1):
            @pl.when(n_win >= d)
            def _():
                g = n_win - d
                for s in range(2):
                    @pl.when(lax.rem(g, 2) == s)
                    def _():
                        wait_out(g, s)

        @pl.when(wid == NW - 1)
        def _():
            v0 = n_full * WC
            pltpu.sync_copy(wt_hbm.at[:, pl.ds(v0, tail_full)],
                            slab0.at[:, pl.ds(0, tail_full)])

            def tb(t8, c):
                _transpose_128(slab0, tr0, t8, t8 * TILE_W, D)
                return c
            lax.fori_loop(0, tail_full // TILE_W, tb, 0)
            pltpu.sync_copy(tr0.at[pl.ds(0, tail_full * D)],
                            g_hbm.at[pl.ds(v0 * D, tail_full * D)])

        @pl.when(wid == NW - 2)
        def _():
            nrem = tail_rem * D
            pltpu.sync_copy(wtail_hbm, tr1.at[pl.ds(0, nrem)])
            pltpu.sync_copy(
                tr1.at[pl.ds(0, nrem)],
                g_hbm.at[pl.ds((n_full * WC + tail_full) * D, nrem)])

    return detile_kernel


@functools.lru_cache(maxsize=None)
def _build_gather(B: int, B_pad: int, V: int, D: int):
    b_per_w = B_pad // NW
    last_w = B - (NW - 1) * b_per_w
    assert b_per_w % 8 == 0 and 0 < last_w <= b_per_w
    mesh = plsc.VectorSubcoreMesh(core_axis_name="c", subcore_axis_name="s")

    @functools.partial(
        pl.kernel,
        mesh=mesh,
        out_type=jax.ShapeDtypeStruct((B, D), jnp.float32),
        scratch_types=[
            pltpu.VMEM((b_per_w,), jnp.int32),
            pltpu.VMEM((b_per_w, D), jnp.float32),
            pltpu.SemaphoreType.DMA,
        ],
        compiler_params=pltpu.CompilerParams(use_tc_tiling_on_sc=False),
    )
    def gather_kernel(idx_hbm, table_hbm, out_hbm, idx_v, rows_v, sem):
        wid = lax.axis_index("s") * NC + lax.axis_index("c")
        base = pl.multiple_of(wid * b_per_w, 8)
        pltpu.sync_copy(idx_hbm.at[pl.ds(base, b_per_w)], idx_v)
        pltpu.async_copy(table_hbm.at[idx_v], rows_v, sem).wait()

        @pl.when(wid < NW - 1)
        def _():
            pltpu.sync_copy(rows_v, out_hbm.at[pl.ds(base, b_per_w), :])

        @pl.when(wid == NW - 1)
        def _():
            pltpu.sync_copy(rows_v.at[pl.ds(0, last_w), :],
                            out_hbm.at[pl.ds(base, last_w), :])

    return gather_kernel


def kernel(atom_types, W):
    idx = jnp.squeeze(atom_types).astype(jnp.int32)
    B = idx.shape[0]
    V, D = W.shape
    align = 16 * NW
    B_pad = ((B + align - 1) // align) * align
    if B_pad != B:
        pad = (jnp.arange(B_pad - B, dtype=jnp.int32) * 997) % V
        idx = jnp.concatenate([idx, pad])
    n_full = V // WC
    tail_full = ((V - n_full * WC) // TILE_W) * TILE_W
    v_aligned = n_full * WC + tail_full
    wtail = W[v_aligned:].reshape(-1)
    table1d = _build_detile(V, D)(W.T, wtail)
    return _build_gather(B, B_pad, V, D)(idx, table1d.reshape(V, D))


# speedup vs baseline: 1.0706x; 1.0209x over previous
import functools

import jax
import jax.numpy as jnp
from jax import lax
from jax.experimental import pallas as pl
from jax.experimental.pallas import tpu as pltpu
from jax.experimental.pallas import tpu_sc as plsc

NC = 2
NS = 16
NW = NC * NS
LANES = 16
TILE_W = 128
WC = 1024
SLABW = 1041


def _iota16():
    return lax.broadcasted_iota(jnp.int32, (LANES,), 0)


def _transpose_128(slab, tr1d, t8, col_base, d):
    rows = _iota16()
    iota = _iota16()

    def gathers(k):
        c0v = jnp.broadcast_to(col_base + k * 16,
                               (LANES,)).astype(jnp.int32)
        return [plsc.load_gather(slab, [rows, c0v + b]) for b in range(16)]

    def stores(k, vals):
        a0v = jnp.broadcast_to((col_base + k * 16) * d,
                               (LANES,)).astype(jnp.int32) + iota
        for b in range(16):
            plsc.store_scatter(tr1d, [a0v + b * d], vals[b])

    def blk(k, vals_prev):
        vals = gathers(k)
        stores(k - 1, vals_prev)
        return vals
    last = lax.fori_loop(1, TILE_W // 16, blk, gathers(0))
    stores(TILE_W // 16 - 1, last)


@functools.lru_cache(maxsize=None)
def _build_detile(V: int, D: int):
    n_full = V // WC
    tail = V - n_full * WC
    tail_full = (tail // TILE_W) * TILE_W
    tail_rem = tail - tail_full
    base_wins = n_full // NW
    extra = n_full - base_wins * NW
    mesh = plsc.VectorSubcoreMesh(core_axis_name="c", subcore_axis_name="s")

    @functools.partial(
        pl.kernel,
        mesh=mesh,
        out_type=jax.ShapeDtypeStruct((V * D,), jnp.float32),
        scratch_types=[
            pltpu.VMEM((LANES, SLABW), jnp.float32),
            pltpu.VMEM((LANES, SLABW), jnp.float32),
            pltpu.VMEM((WC * D,), jnp.float32),
            pltpu.VMEM((WC * D,), jnp.float32),
            pltpu.SemaphoreType.DMA,
            pltpu.SemaphoreType.DMA,
            pltpu.SemaphoreType.DMA,
            pltpu.SemaphoreType.DMA,
        ],
        compiler_params=pltpu.CompilerParams(needs_layout_passes=False),
    )
    def detile_kernel(wt_hbm, wtail_hbm, g_hbm, slab0, slab1, tr0, tr1,
                      si0, si1, so0, so1):
        wid = lax.axis_index("s") * NC + lax.axis_index("c")
        n_win = base_wins + jnp.where(wid < extra, 1, 0)
        slabs = [slab0, slab1]
        trs = [tr0, tr1]
        sis = [si0, si1]
        sos = [so0, so1]

        def v0_of(g):
            return pl.multiple_of((wid + g * NW) * WC, WC)

        def start_in(g, s):
            pltpu.async_copy(wt_hbm.at[:, pl.ds(v0_of(g), WC)],
                             slabs[s].at[:, pl.ds(0, WC)], sis[s])

        def wait_in(g, s):
            pltpu.make_async_copy(wt_hbm.at[:, pl.ds(v0_of(g), WC)],
                                  slabs[s].at[:, pl.ds(0, WC)],
                                  sis[s]).wait()

        def g0_of(g):
            return pl.multiple_of(v0_of(g) * D, WC * D)

        def start_out(g, s):
            pltpu.async_copy(trs[s], g_hbm.at[pl.ds(g0_of(g), WC * D)],
                             sos[s])

        def wait_out(g, s):
            pltpu.make_async_copy(trs[s],
                                  g_hbm.at[pl.ds(g0_of(g), WC * D)],
                                  sos[s]).wait()

        @pl.when(n_win > 0)
        def _():
            start_in(0, 0)

        def body(g, carry):
            for s in range(2):
                @pl.when(lax.rem(g, 2) == s)
                def _():
                    @pl.when(g + 1 < n_win)
                    def _():
                        start_in(g + 1, 1 - s)

                    @pl.when(g >= 2)
                    def _():
                        wait_out(g - 2, s)

                    wait_in(g, s)

                    def tb(t8, c):
                        _transpose_128(slabs[s], trs[s], t8, t8 * TILE_W, D)
                        return c
                    lax.fori_loop(0, WC // TILE_W, tb, 0)
                    start_out(g, s)
            return carry

        lax.fori_loop(0, n_win, body, 0)

        for d in (2, 1):
            @pl.when(n_win >= d)
            def _():
                g = n_win - d
                for s in range(2):
                    @pl.when(lax.rem(g, 2) == s)
                    def _():
                        wait_out(g, s)

        @pl.when(wid == NW - 1)
        def _():
            v0 = n_full * WC
            pltpu.sync_copy(wt_hbm.at[:, pl.ds(v0, tail_full)],
                            slab0.at[:, pl.ds(0, tail_full)])

            def tb(t8, c):
                _transpose_128(slab0, tr0, t8, t8 * TILE_W, D)
                return c
            lax.fori_loop(0, tail_full // TILE_W, tb, 0)
            pltpu.sync_copy(tr0.at[pl.ds(0, tail_full * D)],
                            g_hbm.at[pl.ds(v0 * D, tail_full * D)])

        @pl.when(wid == NW - 2)
        def _():
            nrem = tail_rem * D
            pltpu.sync_copy(wtail_hbm, tr1.at[pl.ds(0, nrem)])
            pltpu.sync_copy(
                tr1.at[pl.ds(0, nrem)],
                g_hbm.at[pl.ds((n_full * WC + tail_full) * D, nrem)])

    return detile_kernel


@functools.lru_cache(maxsize=None)
def _build_gather(B: int, B_pad: int, V: int, D: int):
    b_per_w = B_pad // NW
    last_w = B - (NW - 1) * b_per_w
    assert b_per_w % 8 == 0 and 0 < last_w <= b_per_w
    mesh = plsc.VectorSubcoreMesh(core_axis_name="c", subcore_axis_name="s")

    @functools.partial(
        pl.kernel,
        mesh=mesh,
        out_type=jax.ShapeDtypeStruct((B, D), jnp.float32),
        scratch_types=[
            pltpu.VMEM((b_per_w,), jnp.int32),
            pltpu.VMEM((b_per_w, D), jnp.float32),
            pltpu.SemaphoreType.DMA,
        ],
        compiler_params=pltpu.CompilerParams(use_tc_tiling_on_sc=False),
    )
    def gather_kernel(idx_hbm, table_hbm, out_hbm, idx_v, rows_v, sem):
        wid = lax.axis_index("s") * NC + lax.axis_index("c")
        base = pl.multiple_of(wid * b_per_w, 8)
        pltpu.sync_copy(idx_hbm.at[pl.ds(base, b_per_w)], idx_v)
        pltpu.async_copy(table_hbm.at[idx_v], rows_v, sem).wait()

        @pl.when(wid < NW - 1)
        def _():
            pltpu.sync_copy(rows_v, out_hbm.at[pl.ds(base, b_per_w), :])

        @pl.when(wid == NW - 1)
        def _():
            pltpu.sync_copy(rows_v.at[pl.ds(0, last_w), :],
                            out_hbm.at[pl.ds(base, last_w), :])

    return gather_kernel


def kernel(atom_types, W):
    idx = jnp.squeeze(atom_types).astype(jnp.int32)
    B = idx.shape[0]
    V, D = W.shape
    align = 16 * NW
    B_pad = ((B + align - 1) // align) * align
    if B_pad != B:
        pad = (jnp.arange(B_pad - B, dtype=jnp.int32) * 997) % V
        idx = jnp.concatenate([idx, pad])
    n_full = V // WC
    tail_full = ((V - n_full * WC) // TILE_W) * TILE_W
    v_aligned = n_full * WC + tail_full
    wtail = W[v_aligned:].reshape(-1)
    table1d = _build_detile(V, D)(W.T, wtail)
    return _build_gather(B, B_pad, V, D)(idx, table1d.reshape(V, D))
